# baseline (speedup 1.0000x reference)
import functools

import jax
import jax.numpy as jnp
from jax import lax
from jax.experimental import pallas as pl
from jax.experimental.pallas import tpu as pltpu
from jax.experimental.pallas import tpu_sc as plsc

B = 16384
D = 64
NC = 2
NS = 16
NW = NC * NS
BPW = B // NW
CH = 128
NCH = BPW // CH
L = 16

EH = 1 << 19
TH = 1 << 9
RB = 8192


def _repack_kernel(a_ref, b_ref, o_ref):
    r = lax.broadcasted_iota(jnp.int32, (D, D), 0)
    c = lax.broadcasted_iota(jnp.int32, (D, D), 1)
    eye = (r == c).astype(jnp.float32)
    dn = (((0,), (0,)), ((), ()))
    o_ref[:, 0:D] = lax.dot_general(a_ref[...], eye, dn,
                                    precision=lax.Precision.HIGHEST,
                                    preferred_element_type=jnp.float32)
    o_ref[:, D:2 * D] = lax.dot_general(b_ref[...], eye, dn,
                                        precision=lax.Precision.HIGHEST,
                                        preferred_element_type=jnp.float32)


def _repack(table_t, half, rb):
    grid = half // rb
    n = table_t.shape[1]
    last = (n + rb - 1) // rb - 1
    return pl.pallas_call(
        _repack_kernel,
        grid=(grid,),
        in_specs=[
            pl.BlockSpec((D, rb), lambda i: (0, i)),
            pl.BlockSpec((D, rb),
                         lambda i, g=grid, m=last: (0, jnp.minimum(i + g, m))),
        ],
        out_specs=pl.BlockSpec((rb, 2 * D), lambda i: (i, 0)),
        out_shape=jax.ShapeDtypeStruct((half, 2 * D), jnp.float32),
    )(table_t, table_t)


def _make_dot_kernel():
    mesh = plsc.VectorSubcoreMesh(core_axis_name="c", subcore_axis_name="s")

    @functools.partial(
        pl.kernel,
        mesh=mesh,
        compiler_params=pltpu.CompilerParams(needs_layout_passes=False),
        out_type=jax.ShapeDtypeStruct((B,), jnp.float32),
        scratch_types=[
            pltpu.VMEM((NCH, CH), jnp.int32),
            pltpu.VMEM((NCH, CH), jnp.int32),
            pltpu.VMEM((NCH, CH), jnp.int32),
            pltpu.VMEM((NCH, CH), jnp.int32),
            pltpu.VMEM((2, CH, 2 * D), jnp.float32),
            pltpu.VMEM((2, CH, 2 * D), jnp.float32),
            pltpu.VMEM((BPW,), jnp.float32),
            pltpu.SemaphoreType.DMA,
        ],
    )
    def dot_kernel(ent_hbm, tid_hbm, epk_hbm, tpk_hbm, out_hbm,
                   eidx, tidx, erow, trow, erows, trows, outv, sem):
        wid = lax.axis_index("s") * NC + lax.axis_index("c")
        base = wid * BPW

        for c in range(NCH):
            pltpu.sync_copy(ent_hbm.at[pl.ds(base + c * CH, CH)], eidx.at[c])
            pltpu.sync_copy(tid_hbm.at[pl.ds(base + c * CH, CH)], tidx.at[c])
        for c in range(NCH):
            for j in range(CH // L):
                s = pl.ds(j * L, L)
                erow[c, s] = eidx[c, s] & (EH - 1)
                trow[c, s] = tidx[c, s] & (TH - 1)

        def fire(c):
            eb = erows.at[c % 2]
            tb = trows.at[c % 2]
            return (pltpu.async_copy(epk_hbm.at[erow.at[c]], eb, sem),
                    pltpu.async_copy(tpk_hbm.at[trow.at[c]], tb, sem))

        lane = lax.iota(jnp.int32, L)
        cps = fire(0)

        for c in range(NCH):
            for cp in cps:
                cp.wait()
            if c + 1 < NCH:
                nxt = fire(c + 1)
            eb = erows.at[c % 2]
            tb = trows.at[c % 2]

            def group_body(g, _, c=c, eb=eb, tb=tb):
                rows = g * L + lane
                e16 = eidx[c, pl.ds(g * L, L)]
                t16 = tidx[c, pl.ds(g * L, L)]
                eoff = (e16 >> 19) * D
                toff = (t16 >> 9) * D
                acc = jnp.zeros((L,), jnp.float32)
                for j in range(D):
                    ev = plsc.load_gather(eb, [rows, eoff + j])
                    tv = plsc.load_gather(tb, [rows, toff + j])
                    acc = acc + ev * tv
                outv[pl.ds(c * CH + g * L, L)] = acc
                return 0

            lax.fori_loop(0, CH // L, group_body, 0)
            if c + 1 < NCH:
                cps = nxt

        pltpu.sync_copy(outv, out_hbm.at[pl.ds(base, BPW)])

    return dot_kernel


_dot_kernel = _make_dot_kernel()


def kernel(ent, ent_type, batch_type, ent_emb, type_emb):
    del batch_type
    epk = _repack(ent_emb.T, EH, RB)
    tpk = _repack(type_emb.T, TH, TH)
    score = _dot_kernel(ent.astype(jnp.int32), ent_type.astype(jnp.int32),
                        epk, tpk)
    return score[:, None]


# speedup vs baseline: 1.9427x; 1.9427x over previous
import functools

import jax
import jax.numpy as jnp
from jax import lax
from jax.experimental import pallas as pl
from jax.experimental.pallas import tpu as pltpu
from jax.experimental.pallas import tpu_sc as plsc

B = 16384
D = 64
NC = 2
NS = 16
NW = NC * NS
BPW = B // NW
CH = 128
NCH = BPW // CH
L = 16

EH = 1 << 19
TH = 1 << 9
RB = 8192


def _repack_kernel(a_ref, b_ref, o_ref):
    r = lax.broadcasted_iota(jnp.int32, (D, D), 0)
    c = lax.broadcasted_iota(jnp.int32, (D, D), 1)
    eye = (r == c).astype(jnp.float32)
    dn = (((0,), (0,)), ((), ()))
    o_ref[:, 0:D] = lax.dot_general(a_ref[...], eye, dn,

                                    preferred_element_type=jnp.float32)
    o_ref[:, D:2 * D] = lax.dot_general(b_ref[...], eye, dn,

                                        preferred_element_type=jnp.float32)


def _repack(table_t, half, rb):
    grid = half // rb
    n = table_t.shape[1]
    last = (n + rb - 1) // rb - 1
    return pl.pallas_call(
        _repack_kernel,
        grid=(grid,),
        in_specs=[
            pl.BlockSpec((D, rb), lambda i: (0, i)),
            pl.BlockSpec((D, rb),
                         lambda i, g=grid, m=last: (0, jnp.minimum(i + g, m))),
        ],
        out_specs=pl.BlockSpec((rb, 2 * D), lambda i: (i, 0)),
        out_shape=jax.ShapeDtypeStruct((half, 2 * D), jnp.float32),
    )(table_t, table_t)


def _make_dot_kernel():
    mesh = plsc.VectorSubcoreMesh(core_axis_name="c", subcore_axis_name="s")

    @functools.partial(
        pl.kernel,
        mesh=mesh,
        compiler_params=pltpu.CompilerParams(needs_layout_passes=False),
        out_type=jax.ShapeDtypeStruct((B,), jnp.float32),
        scratch_types=[
            pltpu.VMEM((NCH, CH), jnp.int32),
            pltpu.VMEM((NCH, CH), jnp.int32),
            pltpu.VMEM((NCH, CH), jnp.int32),
            pltpu.VMEM((NCH, CH), jnp.int32),
            pltpu.VMEM((2, CH, 2 * D), jnp.float32),
            pltpu.VMEM((2, CH, 2 * D), jnp.float32),
            pltpu.VMEM((BPW,), jnp.float32),
            pltpu.SemaphoreType.DMA,
        ],
    )
    def dot_kernel(ent_hbm, tid_hbm, epk_hbm, tpk_hbm, out_hbm,
                   eidx, tidx, erow, trow, erows, trows, outv, sem):
        wid = lax.axis_index("s") * NC + lax.axis_index("c")
        base = wid * BPW

        for c in range(NCH):
            pltpu.sync_copy(ent_hbm.at[pl.ds(base + c * CH, CH)], eidx.at[c])
            pltpu.sync_copy(tid_hbm.at[pl.ds(base + c * CH, CH)], tidx.at[c])
        for c in range(NCH):
            for j in range(CH // L):
                s = pl.ds(j * L, L)
                erow[c, s] = eidx[c, s] & (EH - 1)
                trow[c, s] = tidx[c, s] & (TH - 1)

        def fire(c):
            eb = erows.at[c % 2]
            tb = trows.at[c % 2]
            return (pltpu.async_copy(epk_hbm.at[erow.at[c]], eb, sem),
                    pltpu.async_copy(tpk_hbm.at[trow.at[c]], tb, sem))

        lane = lax.iota(jnp.int32, L)
        cps = fire(0)

        for c in range(NCH):
            for cp in cps:
                cp.wait()
            if c + 1 < NCH:
                nxt = fire(c + 1)
            eb = erows.at[c % 2]
            tb = trows.at[c % 2]

            def group_body(g, _, c=c, eb=eb, tb=tb):
                rows = g * L + lane
                e16 = eidx[c, pl.ds(g * L, L)]
                t16 = tidx[c, pl.ds(g * L, L)]
                eoff = (e16 >> 19) * D
                toff = (t16 >> 9) * D
                acc = jnp.zeros((L,), jnp.float32)
                for j in range(D):
                    ev = plsc.load_gather(eb, [rows, eoff + j])
                    tv = plsc.load_gather(tb, [rows, toff + j])
                    acc = acc + ev * tv
                outv[pl.ds(c * CH + g * L, L)] = acc
                return 0

            lax.fori_loop(0, CH // L, group_body, 0)
            if c + 1 < NCH:
                cps = nxt

        pltpu.sync_copy(outv, out_hbm.at[pl.ds(base, BPW)])

    return dot_kernel


_dot_kernel = _make_dot_kernel()


def kernel(ent, ent_type, batch_type, ent_emb, type_emb):
    del batch_type
    epk = _repack(ent_emb.T, EH, RB)
    tpk = _repack(type_emb.T, TH, TH)
    score = _dot_kernel(ent.astype(jnp.int32), ent_type.astype(jnp.int32),
                        epk, tpk)
    return score[:, None]


# speedup vs baseline: 1.9487x; 1.0031x over previous
import functools

import jax
import jax.numpy as jnp
from jax import lax
from jax.experimental import pallas as pl
from jax.experimental.pallas import tpu as pltpu
from jax.experimental.pallas import tpu_sc as plsc

B = 16384
D = 64
NC = 2
NS = 16
NW = NC * NS
BPW = B // NW
CH = 128
NCH = BPW // CH
L = 16

EH = 1 << 19
TH = 1 << 9
RB = 8192


def _repack_kernel(a_ref, b_ref, o_ref):
    o_ref[:, 0:D] = jnp.swapaxes(a_ref[...], 0, 1)
    o_ref[:, D:2 * D] = jnp.swapaxes(b_ref[...], 0, 1)


def _repack(table_t, half, rb):
    grid = half // rb
    n = table_t.shape[1]
    last = (n + rb - 1) // rb - 1
    return pl.pallas_call(
        _repack_kernel,
        grid=(grid,),
        in_specs=[
            pl.BlockSpec((D, rb), lambda i: (0, i)),
            pl.BlockSpec((D, rb),
                         lambda i, g=grid, m=last: (0, jnp.minimum(i + g, m))),
        ],
        out_specs=pl.BlockSpec((rb, 2 * D), lambda i: (i, 0)),
        out_shape=jax.ShapeDtypeStruct((half, 2 * D), jnp.float32),
    )(table_t, table_t)


def _make_dot_kernel():
    mesh = plsc.VectorSubcoreMesh(core_axis_name="c", subcore_axis_name="s")

    @functools.partial(
        pl.kernel,
        mesh=mesh,
        compiler_params=pltpu.CompilerParams(needs_layout_passes=False,
                                             use_tc_tiling_on_sc=False),
        out_type=jax.ShapeDtypeStruct((B,), jnp.float32),
        scratch_types=[
            pltpu.VMEM((NCH, CH), jnp.int32),
            pltpu.VMEM((NCH, CH), jnp.int32),
            pltpu.VMEM((NCH, CH), jnp.int32),
            pltpu.VMEM((NCH, CH), jnp.int32),
            pltpu.VMEM((2, CH, D), jnp.float32),
            pltpu.VMEM((2, CH, D), jnp.float32),
            pltpu.VMEM((BPW,), jnp.float32),
            pltpu.SemaphoreType.DMA,
        ],
    )
    def dot_kernel(ent_hbm, tid_hbm, epk_hbm, tpk_hbm, out_hbm,
                   eidx, tidx, erow, trow, erows, trows, outv, sem):
        wid = lax.axis_index("s") * NC + lax.axis_index("c")
        base = wid * BPW

        for c in range(NCH):
            pltpu.sync_copy(ent_hbm.at[pl.ds(base + c * CH, CH)], eidx.at[c])
            pltpu.sync_copy(tid_hbm.at[pl.ds(base + c * CH, CH)], tidx.at[c])
        for c in range(NCH):
            for j in range(CH // L):
                s = pl.ds(j * L, L)
                erow[c, s] = ((eidx[c, s] & (EH - 1)) << 1) | (eidx[c, s] >> 19)
                trow[c, s] = ((tidx[c, s] & (TH - 1)) << 1) | (tidx[c, s] >> 9)

        def fire(c):
            eb = erows.at[c % 2]
            tb = trows.at[c % 2]
            return (pltpu.async_copy(epk_hbm.at[erow.at[c]], eb, sem),
                    pltpu.async_copy(tpk_hbm.at[trow.at[c]], tb, sem))

        lane = lax.iota(jnp.int32, L)
        cps = fire(0)

        for c in range(NCH):
            for cp in cps:
                cp.wait()
            if c + 1 < NCH:
                nxt = fire(c + 1)
            eb = erows.at[c % 2]
            tb = trows.at[c % 2]

            def group_body(g, _, c=c, eb=eb, tb=tb):
                rows = g * L + lane
                acc = jnp.zeros((L,), jnp.float32)
                for j in range(D):
                    col = jnp.full((L,), j, jnp.int32)
                    ev = plsc.load_gather(eb, [rows, col])
                    tv = plsc.load_gather(tb, [rows, col])
                    acc = acc + ev * tv
                outv[pl.ds(c * CH + g * L, L)] = acc
                return 0

            lax.fori_loop(0, CH // L, group_body, 0)
            if c + 1 < NCH:
                cps = nxt

        pltpu.sync_copy(outv, out_hbm.at[pl.ds(base, BPW)])

    return dot_kernel


_dot_kernel = _make_dot_kernel()


def kernel(ent, ent_type, batch_type, ent_emb, type_emb):
    del batch_type
    epk = _repack(ent_emb.T, EH, RB).reshape(2 * EH, D)
    tpk = _repack(type_emb.T, TH, TH).reshape(2 * TH, D)
    score = _dot_kernel(ent.astype(jnp.int32), ent_type.astype(jnp.int32),
                        epk, tpk)
    return score[:, None]


# speedup vs baseline: 2.1562x; 1.1065x over previous
import functools

import jax
import jax.numpy as jnp
from jax import lax
from jax.experimental import pallas as pl
from jax.experimental.pallas import tpu as pltpu
from jax.experimental.pallas import tpu_sc as plsc

B = 16384
D = 64
NC = 2
NS = 16
NW = NC * NS
BPW = B // NW
CH = 128
NCH = BPW // CH
L = 16

EH = 1 << 19
TH = 1 << 9
RB = 8192


def _repack_kernel(a_ref, b_ref, o_ref):
    o_ref[:, 0:D] = jnp.swapaxes(a_ref[...], 0, 1)
    o_ref[:, D:2 * D] = jnp.swapaxes(b_ref[...], 0, 1)


def _repack(table_t, half, rb):
    grid = half // rb
    n = table_t.shape[1]
    last = (n + rb - 1) // rb - 1
    return pl.pallas_call(
        _repack_kernel,
        grid=(grid,),
        in_specs=[
            pl.BlockSpec((D, rb), lambda i: (0, i)),
            pl.BlockSpec((D, rb),
                         lambda i, g=grid, m=last: (0, jnp.minimum(i + g, m))),
        ],
        out_specs=pl.BlockSpec((rb, 2 * D), lambda i: (i, 0)),
        out_shape=jax.ShapeDtypeStruct((half, 2 * D), jnp.float32),
    )(table_t, table_t)


def _make_dot_kernel():
    mesh = plsc.VectorSubcoreMesh(core_axis_name="c", subcore_axis_name="s")

    @functools.partial(
        pl.kernel,
        mesh=mesh,
        compiler_params=pltpu.CompilerParams(needs_layout_passes=False,
                                             use_tc_tiling_on_sc=False),
        out_type=jax.ShapeDtypeStruct((B,), jnp.float32),
        scratch_types=[
            pltpu.VMEM((NCH, CH), jnp.int32),
            pltpu.VMEM((NCH, CH), jnp.int32),
            pltpu.VMEM((NCH, CH), jnp.int32),
            pltpu.VMEM((NCH, CH), jnp.int32),
            pltpu.VMEM((BPW, D), jnp.float32),
            pltpu.VMEM((BPW, D), jnp.float32),
            pltpu.VMEM((BPW,), jnp.float32),
            pltpu.SemaphoreType.DMA,
        ],
    )
    def dot_kernel(ent_hbm, tid_hbm, epk_hbm, tpk_hbm, out_hbm,
                   eidx, tidx, erow, trow, erows, trows, outv, sem):
        wid = lax.axis_index("s") * NC + lax.axis_index("c")
        base = wid * BPW

        for c in range(NCH):
            pltpu.sync_copy(ent_hbm.at[pl.ds(base + c * CH, CH)], eidx.at[c])
            pltpu.sync_copy(tid_hbm.at[pl.ds(base + c * CH, CH)], tidx.at[c])
        for c in range(NCH):
            for j in range(CH // L):
                s = pl.ds(j * L, L)
                e = eidx[c, s]
                t = tidx[c, s]
                erow[c, s] = ((e & (EH - 1)) << 1) | (e >> 19)
                trow[c, s] = ((t & (TH - 1)) << 1) | (t >> 9)

        cps = []
        for c in range(NCH):
            cps.append(pltpu.async_copy(
                epk_hbm.at[erow.at[c]], erows.at[pl.ds(c * CH, CH), :], sem))
            cps.append(pltpu.async_copy(
                tpk_hbm.at[trow.at[c]], trows.at[pl.ds(c * CH, CH), :], sem))
        for cp in cps:
            cp.wait()

        lane = lax.iota(jnp.int32, L)

        def group_body(g, _):
            rbase = g * L
            acc = jnp.zeros((L,), jnp.float32)
            for i in range(L):
                r = rbase + i
                p = erows[r, pl.ds(0, L)] * trows[r, pl.ds(0, L)]
                for k in range(1, D // L):
                    p = p + erows[r, pl.ds(k * L, L)] * trows[r, pl.ds(k * L, L)]
                acc = jnp.where(lane == i, jnp.sum(p), acc)
            outv[pl.ds(rbase, L)] = acc
            return 0

        lax.fori_loop(0, BPW // L, group_body, 0)

        pltpu.sync_copy(outv, out_hbm.at[pl.ds(base, BPW)])

    return dot_kernel


_dot_kernel = _make_dot_kernel()


def kernel(ent, ent_type, batch_type, ent_emb, type_emb):
    del batch_type
    epk = _repack(ent_emb.T, EH, RB).reshape(2 * EH, D)
    tpk = _repack(type_emb.T, TH, TH).reshape(2 * TH, D)
    score = _dot_kernel(ent.astype(jnp.int32), ent_type.astype(jnp.int32),
                        epk, tpk)
    return score[:, None]


# speedup vs baseline: 2.2879x; 1.0611x over previous
import functools

import jax
import jax.numpy as jnp
from jax import lax
from jax.experimental import pallas as pl
from jax.experimental.pallas import tpu as pltpu
from jax.experimental.pallas import tpu_sc as plsc

B = 16384
D = 64
NC = 2
NS = 16
NW = NC * NS
BPW = B // NW
CH = 128
NCH = BPW // CH
L = 16

EH = 1 << 19
TH = 1 << 9
RB = 16384


def _repack_kernel(a_ref, b_ref, o_ref):
    o_ref[:, 0:D] = jnp.swapaxes(a_ref[...], 0, 1)
    o_ref[:, D:2 * D] = jnp.swapaxes(b_ref[...], 0, 1)


def _repack(table_t, half, rb):
    grid = half // rb
    n = table_t.shape[1]
    last = (n + rb - 1) // rb - 1
    return pl.pallas_call(
        _repack_kernel,
        grid=(grid,),
        in_specs=[
            pl.BlockSpec((D, rb), lambda i: (0, i)),
            pl.BlockSpec((D, rb),
                         lambda i, g=grid, m=last: (0, jnp.minimum(i + g, m))),
        ],
        out_specs=pl.BlockSpec((rb, 2 * D), lambda i: (i, 0)),
        out_shape=jax.ShapeDtypeStruct((half, 2 * D), jnp.float32),
    )(table_t, table_t)


def _make_dot_kernel():
    mesh = plsc.VectorSubcoreMesh(core_axis_name="c", subcore_axis_name="s")

    @functools.partial(
        pl.kernel,
        mesh=mesh,
        compiler_params=pltpu.CompilerParams(needs_layout_passes=False,
                                             use_tc_tiling_on_sc=False),
        out_type=jax.ShapeDtypeStruct((B,), jnp.float32),
        scratch_types=[
            pltpu.VMEM((NCH, CH), jnp.int32),
            pltpu.VMEM((NCH, CH), jnp.int32),
            pltpu.VMEM((NCH, CH), jnp.int32),
            pltpu.VMEM((NCH, CH), jnp.int32),
            pltpu.VMEM((BPW, D), jnp.float32),
            pltpu.VMEM((BPW, D), jnp.float32),
            pltpu.VMEM((BPW,), jnp.float32),
            pltpu.SemaphoreType.DMA,
        ],
    )
    def dot_kernel(ent_hbm, tid_hbm, epk_hbm, tpk_hbm, out_hbm,
                   eidx, tidx, erow, trow, erows, trows, outv, sem):
        wid = lax.axis_index("s") * NC + lax.axis_index("c")
        base = wid * BPW

        for c in range(NCH):
            pltpu.sync_copy(ent_hbm.at[pl.ds(base + c * CH, CH)], eidx.at[c])
            pltpu.sync_copy(tid_hbm.at[pl.ds(base + c * CH, CH)], tidx.at[c])
        for c in range(NCH):
            for j in range(CH // L):
                s = pl.ds(j * L, L)
                e = eidx[c, s]
                t = tidx[c, s]
                erow[c, s] = ((e & (EH - 1)) << 1) | (e >> 19)
                trow[c, s] = ((t & (TH - 1)) << 1) | (t >> 9)

        cps = []
        for c in range(NCH):
            cps.append(pltpu.async_copy(
                epk_hbm.at[erow.at[c]], erows.at[pl.ds(c * CH, CH), :], sem))
            cps.append(pltpu.async_copy(
                tpk_hbm.at[trow.at[c]], trows.at[pl.ds(c * CH, CH), :], sem))
        for cp in cps:
            cp.wait()

        lane = lax.iota(jnp.int32, L)

        def group_body(g, _):
            rbase = g * L
            acc = jnp.zeros((L,), jnp.float32)
            for i in range(L):
                r = rbase + i
                p = erows[r, pl.ds(0, L)] * trows[r, pl.ds(0, L)]
                for k in range(1, D // L):
                    p = p + erows[r, pl.ds(k * L, L)] * trows[r, pl.ds(k * L, L)]
                acc = jnp.where(lane == i, jnp.sum(p), acc)
            outv[pl.ds(rbase, L)] = acc
            return 0

        lax.fori_loop(0, BPW // L, group_body, 0)

        pltpu.sync_copy(outv, out_hbm.at[pl.ds(base, BPW)])

    return dot_kernel


_dot_kernel = _make_dot_kernel()


def kernel(ent, ent_type, batch_type, ent_emb, type_emb):
    del batch_type
    epk = _repack(ent_emb.T, EH, RB).reshape(2 * EH, D)
    tpk = _repack(type_emb.T, TH, TH).reshape(2 * TH, D)
    score = _dot_kernel(ent.astype(jnp.int32), ent_type.astype(jnp.int32),
                        epk, tpk)
    return score[:, None]
